# baseline (speedup 1.0000x reference)
import functools

import jax
import jax.numpy as jnp
from jax import lax
from jax.experimental import pallas as pl
from jax.experimental.pallas import tpu as pltpu
from jax.experimental.pallas import tpu_sc as plsc

_NC = 2
_NS = 16
_NW = _NC * _NS
_D = 128
_CHUNK = 128
_LANES = 16
_NBUF = 5
_NSTREAM = 4


def kernel(resource_ids, id_embedding):
    n_rows, n_cols = resource_ids.shape
    B = n_rows * n_cols
    n_per_w = B // _NW
    n_chunks = n_per_w // _CHUNK
    n_outer = n_chunks // _NBUF
    ids = resource_ids.reshape(_NW, n_chunks, _CHUNK)

    mesh = plsc.VectorSubcoreMesh(core_axis_name="c", subcore_axis_name="s")

    @functools.partial(
        pl.kernel,
        mesh=mesh,
        out_type=jax.ShapeDtypeStruct((B, _D), jnp.float32),
        scratch_types=[
            pltpu.VMEM((n_chunks, _CHUNK), jnp.int32),
            pltpu.VMEM_SHARED((4, _D), jnp.float32),
            pltpu.VMEM((4, _D), jnp.float32),
        ]
        + [pltpu.VMEM((_CHUNK, _D), jnp.float32) for _ in range(_NBUF)]
        + [pltpu.SemaphoreType.DMA for _ in range(_NSTREAM + _NBUF)],
        compiler_params=pltpu.CompilerParams(needs_layout_passes=False),
    )
    def _emb(ids_hbm, tab_hbm, out_hbm, idx_v, tab_s, tab_v, *bufs_sems):
        rows = bufs_sems[:_NBUF]
        gsems = bufs_sems[_NBUF : _NBUF + _NSTREAM]
        osems = bufs_sems[_NBUF + _NSTREAM :]
        sid = lax.axis_index("s")
        wid = sid * _NC + lax.axis_index("c")

        @pl.when(sid == 0)
        def _stage_table():
            pltpu.sync_copy(tab_hbm, tab_s)

        pltpu.sync_copy(tab_hbm, tab_v)
        pltpu.sync_copy(ids_hbm.at[wid], idx_v)

        def clamp_row(g, carry):
            row = idx_v.at[g]
            for j in range(_CHUNK // _LANES):
                sl = pl.ds(j * _LANES, _LANES)
                row[sl] = jnp.minimum(row[sl], 3)
            return carry

        lax.fori_loop(0, n_chunks, clamp_row, 0)
        plsc.subcore_barrier()

        out_base = wid * n_per_w
        offs = [j * _LANES + lax.iota(jnp.int32, _LANES) for j in range(_D // _LANES)]

        def drain(sem, buf):
            pltpu.make_async_copy(out_hbm.at[pl.ds(out_base, _CHUNK)], buf, sem).wait()

        def start_out(g, b):
            pltpu.async_copy(
                rows[b], out_hbm.at[pl.ds(out_base + g * _CHUNK, _CHUNK)], osems[b]
            )

        def outer(t, carry):
            for b in range(_NSTREAM):
                g = t * _NBUF + b

                @pl.when(t >= 1)
                def _w():
                    drain(osems[b], rows[b])

                pltpu.async_copy(tab_s.at[idx_v.at[g]], rows[b], gsems[b])

            bv_ = _NBUF - 1
            gv = t * _NBUF + bv_

            @pl.when(t >= 1)
            def _wv():
                drain(osems[bv_], rows[bv_])

            rbuf = rows[bv_]

            def _expand(i16, carry):
                bv = idx_v[gv, pl.ds(i16 * _LANES, _LANES)]
                for l in range(_LANES):
                    rowv = jnp.broadcast_to(bv[l], (_LANES,))
                    row = rbuf.at[i16 * _LANES + l]
                    for j in range(_D // _LANES):
                        row[pl.ds(j * _LANES, _LANES)] = plsc.load_gather(
                            tab_v, [rowv, offs[j]]
                        )
                return carry

            lax.fori_loop(0, _CHUNK // _LANES, _expand, 0)

            start_out(gv, bv_)
            for b in range(_NSTREAM):
                drain(gsems[b], rows[b])
                start_out(t * _NBUF + b, b)
            return carry

        lax.fori_loop(0, n_outer, outer, 0)
        for b in range(_NBUF):
            drain(osems[b], rows[b])

    out = _emb(ids, id_embedding)
    return out.reshape(n_rows, n_cols, _D)


# speedup vs baseline: 1.3685x; 1.3685x over previous
import functools

import jax
import jax.numpy as jnp
from jax import lax
from jax.experimental import pallas as pl
from jax.experimental.pallas import tpu as pltpu
from jax.experimental.pallas import tpu_sc as plsc

_NC = 2
_NS = 16
_NW = _NC * _NS
_D = 128
_Q = 2
_QD = _Q * _D
_IDXW = 128
_DESC = 128
_LANES = 16
_NBUF = 2
_AHEAD = 1


def kernel(resource_ids, id_embedding):
    n_rows, n_cols = resource_ids.shape
    B = n_rows * n_cols
    nq = B // _Q
    n_per_w = nq // _NW
    n_irows = (n_per_w * _Q) // _IDXW
    n_desc = n_per_w // _DESC
    n_outer = n_desc // _NBUF
    ids = resource_ids.reshape(_NW, n_irows, _IDXW)

    qi = jnp.arange(16, dtype=jnp.int32)
    qtab = jnp.concatenate(
        [jnp.take(id_embedding, (qi >> s) & 3, axis=0) for s in (2, 0)],
        axis=1,
    ).reshape(16, _Q, _D)

    mesh = plsc.VectorSubcoreMesh(core_axis_name="c", subcore_axis_name="s")

    @functools.partial(
        pl.kernel,
        mesh=mesh,
        out_type=jax.ShapeDtypeStruct((nq, _Q, _D), jnp.float32),
        scratch_types=[
            pltpu.VMEM((n_irows, _IDXW), jnp.int32),
            pltpu.VMEM_SHARED((16, _Q, _D), jnp.float32),
        ]
        + [pltpu.VMEM((_DESC, _Q, _D), jnp.float32) for _ in range(_NBUF)]
        + [pltpu.SemaphoreType.DMA for _ in range(2 * _NBUF)],
        compiler_params=pltpu.CompilerParams(needs_layout_passes=False),
    )
    def _emb(ids_hbm, qtab_hbm, out_hbm, idx_v, tab_s, *bufs_sems):
        rows = bufs_sems[:_NBUF]
        gsems = bufs_sems[_NBUF : 2 * _NBUF]
        osems = bufs_sems[2 * _NBUF :]
        sid = lax.axis_index("s")
        wid = sid * _NC + lax.axis_index("c")

        @pl.when(sid == 0)
        def _stage_table():
            pltpu.sync_copy(qtab_hbm, tab_s)

        pltpu.sync_copy(ids_hbm.at[wid], idx_v)

        def clamp_row(r, carry):
            row = idx_v.at[r]
            for j in range(_IDXW // _LANES):
                sl = pl.ds(j * _LANES, _LANES)
                row[sl] = jnp.minimum(row[sl], 3)
            return carry

        lax.fori_loop(0, n_irows, clamp_row, 0)

        def pack_row(d, carry):
            iot = lax.iota(jnp.int32, _LANES)
            dst = idx_v.at[_Q * d]
            for h in range(_IDXW // _LANES):
                src = idx_v.at[_Q * d + (h * _LANES * _Q) // _IDXW]
                cbase = (h * _LANES * _Q) % _IDXW
                acc = None
                for k in range(_Q):
                    v = plsc.load_gather(src, [cbase + iot * _Q + k])
                    acc = v if acc is None else (acc * 4 + v)
                dst[pl.ds(h * _LANES, _LANES)] = acc
            return carry

        lax.fori_loop(0, n_desc, pack_row, 0)
        plsc.subcore_barrier()

        out_base = wid * n_per_w

        def start_gather(d, b):
            pltpu.async_copy(tab_s.at[idx_v.at[_Q * d]], rows[b], gsems[b])

        def start_out(d, b):
            pltpu.async_copy(
                rows[b], out_hbm.at[pl.ds(out_base + d * _DESC, _DESC)], osems[b]
            )

        def drain(sem, buf):
            pltpu.make_async_copy(out_hbm.at[pl.ds(out_base, _DESC)], buf, sem).wait()

        for a in range(_AHEAD):
            start_gather(a, a)

        def outer(t, carry):
            for b in range(_NBUF):
                d = t * _NBUF + b
                drain(gsems[b], rows[b])
                start_out(d, b)
                b2 = (b + _AHEAD) % _NBUF
                dnxt = d + _AHEAD

                def _prefetch():
                    drain(osems[b2], rows[b2])
                    start_gather(dnxt, b2)

                def _prefetch_first():
                    start_gather(dnxt, b2)

                if b + _AHEAD < _NBUF:

                    @pl.when(t == 0)
                    def _p0():
                        _prefetch_first()

                    @pl.when(t >= 1)
                    def _p1():
                        _prefetch()
                else:

                    @pl.when(t < n_outer - 1)
                    def _p2():
                        _prefetch()
            return carry

        lax.fori_loop(0, n_outer, outer, 0)
        for d in range(n_desc - _NBUF, n_desc):
            drain(osems[d % _NBUF], rows[d % _NBUF])

    out = _emb(ids, qtab)
    return out.reshape(n_rows, n_cols, _D)
